# baseline (speedup 1.0000x reference)
import functools

import jax
import jax.numpy as jnp
from jax import lax
from jax.experimental import pallas as pl
from jax.experimental.pallas import tpu as pltpu
from jax.experimental.pallas import tpu_sc as plsc

N = 4096
NFEAT = 512
TDIM = 128
NCLASS = 64
FANOUT = 4

R = 128


def _gather_sc(table, idx):
    B = idx.shape[0]
    D = table.shape[1]
    info = plsc.get_sparse_core_info()
    nw = info.num_cores * info.num_subcores
    b_per_w = B // nw
    mesh = plsc.VectorSubcoreMesh(core_axis_name="c", subcore_axis_name="s")

    @functools.partial(
        pl.kernel,
        mesh=mesh,
        out_type=jax.ShapeDtypeStruct((B, D), jnp.float32),
        scratch_types=[
            pltpu.VMEM((b_per_w,), jnp.int32),
            pltpu.VMEM((b_per_w, D), jnp.float32),
            pltpu.SemaphoreType.DMA,
        ],
    )
    def k(table_hbm, idx_hbm, out_hbm, idx_v, rows_v, sem):
        wid = lax.axis_index("s") * info.num_cores + lax.axis_index("c")
        base = wid * b_per_w
        pltpu.sync_copy(idx_hbm.at[pl.ds(base, b_per_w)], idx_v)
        pltpu.async_copy(table_hbm.at[idx_v], rows_v, sem).wait()
        pltpu.sync_copy(rows_v, out_hbm.at[pl.ds(base, b_per_w)])

    return k(table, idx)


def _mm(a, b):
    return lax.dot_general(a, b, (((1,), (0,)), ((), ())),
                           preferred_element_type=jnp.float32)


def _mmb(a, b):
    return lax.dot_general(a.astype(jnp.bfloat16), b.astype(jnp.bfloat16),
                           (((1,), (0,)), ((), ())),
                           preferred_element_type=jnp.float32)


def _mean4_via(scratch, x):
    d = x.shape[1]
    cols = []
    for c in range(d // 128):
        s = scratch[c]
        s[...] = x[:, c * 128:(c + 1) * 128]
        cols.append(s[0::4] + s[1::4] + s[2::4] + s[3::4])
    m = cols[0] if len(cols) == 1 else jnp.concatenate(cols, axis=1)
    return m * 0.25


def _tc_body(x0_r, x1_r, x2_r, xb0_r, e1_r, xb2_r, wt_r,
             w0a_r, w0b_r, b0_r, w1a_r, w1b_r, b1_r,
             wb0a_r, wb0b_r, bb0_r, wb1a_r, wb1b_r, bb1_r,
             lwa_r, lwb_r, lb_r, out_r,
             s1a, s1b, s1c, s1d, s2a, s2b, s2c, s2d,
             sha, shb, se2, sg1):
    relu = jax.nn.relu
    w0a, w0b, b0 = w0a_r[...], w0b_r[...], b0_r[...]
    wb0a, wb0b, bb0 = wb0a_r[...], wb0b_r[...], bb0_r[...]
    wt = wt_r[...]

    x1 = x1_r[...]
    m1 = _mean4_via([s1a, s1b, s1c, s1d], x1)
    h0 = relu(_mmb(x0_r[...], w0a) + _mmb(m1, w0b) + b0)
    m2 = _mean4_via([s2a, s2b, s2c, s2d], x2_r[...])
    h1 = relu(_mmb(x1, w0a) + _mmb(m2, w0b) + b0)
    mh1 = _mean4_via([sha, shb], h1)
    o1 = relu(_mmb(h0, w1a_r[...]) + _mmb(mh1, w1b_r[...]) + b1_r[...])

    e0 = relu(_mmb(xb0_r[...], wt))
    me1 = (e1_r[0::4] + e1_r[1::4] + e1_r[2::4] + e1_r[3::4]) * 0.25
    g0 = relu(_mmb(e0, wb0a) + _mmb(me1, wb0b) + bb0)
    e1 = e1_r[...]
    e2 = relu(_mmb(xb2_r[...], wt))
    me2 = _mean4_via([se2], e2)
    g1 = relu(_mmb(e1, wb0a) + _mmb(me2, wb0b) + bb0)
    mg1 = _mean4_via([sg1], g1)
    o2 = relu(_mmb(g0, wb1a_r[...]) + _mmb(mg1, wb1b_r[...]) + bb1_r[...])

    z = _mm(o1, lwa_r[...]) + _mm(o2, lwb_r[...]) + lb_r[...]
    zs = z - jnp.max(z, axis=1, keepdims=True)
    out_r[...] = zs - jnp.log(jnp.sum(jnp.exp(zs), axis=1, keepdims=True))


def kernel(x0, x1, x2, x_bi0, x_bi1, x_bi2, table, weight_trans,
           W0, b0, W1, b1, Wb0, bb0, Wb1, bb1, lin_W, lin_b):
    e1 = _gather_sc(table, x_bi1.astype(jnp.int32))

    W0a, W0b = W0[:NFEAT], W0[NFEAT:]
    W1a, W1b = W1[:256], W1[256:]
    Wb0a, Wb0b = Wb0[:TDIM], Wb0[TDIM:]
    Wb1a, Wb1b = Wb1[:128], Wb1[128:]
    lwa, lwb = lin_W[:NCLASS], lin_W[NCLASS:]
    b0_ = b0.reshape(1, -1)
    b1_ = b1.reshape(1, -1)
    bb0_ = bb0.reshape(1, -1)
    bb1_ = bb1.reshape(1, -1)
    lb_ = lin_b.reshape(1, -1)

    nb = N // R
    full = lambda shp: pl.BlockSpec(shp, lambda i: (0, 0))
    row = lambda r, d: pl.BlockSpec((r, d), lambda i: (i, 0))
    out = pl.pallas_call(
        _tc_body,
        grid=(nb,),
        in_specs=[
            row(R, NFEAT),
            row(4 * R, NFEAT),
            row(16 * R, NFEAT),
            row(R, NFEAT),
            row(4 * R, TDIM),
            row(16 * R, NFEAT),
            full((NFEAT, TDIM)),
            full((NFEAT, 256)),
            full((NFEAT, 256)),
            full((1, 256)),
            full((256, NCLASS)),
            full((256, NCLASS)),
            full((1, NCLASS)),
            full((TDIM, 128)),
            full((TDIM, 128)),
            full((1, 128)),
            full((128, 128)),
            full((128, 128)),
            full((1, 128)),
            full((NCLASS, NCLASS)),
            full((128, NCLASS)),
            full((1, NCLASS)),
        ],
        out_specs=row(R, NCLASS),
        out_shape=jax.ShapeDtypeStruct((N, NCLASS), jnp.float32),
        scratch_shapes=[
            pltpu.VMEM((4 * R, 128), jnp.float32),
            pltpu.VMEM((4 * R, 128), jnp.float32),
            pltpu.VMEM((4 * R, 128), jnp.float32),
            pltpu.VMEM((4 * R, 128), jnp.float32),
            pltpu.VMEM((16 * R, 128), jnp.float32),
            pltpu.VMEM((16 * R, 128), jnp.float32),
            pltpu.VMEM((16 * R, 128), jnp.float32),
            pltpu.VMEM((16 * R, 128), jnp.float32),
            pltpu.VMEM((4 * R, 128), jnp.float32),
            pltpu.VMEM((4 * R, 128), jnp.float32),
            pltpu.VMEM((16 * R, 128), jnp.float32),
            pltpu.VMEM((4 * R, 128), jnp.float32),
        ],
    )(x0, x1, x2, x_bi0, e1, x_bi2, weight_trans,
      W0a, W0b, b0_, W1a, W1b, b1_, Wb0a, Wb0b, bb0_,
      Wb1a, Wb1b, bb1_, lwa, lwb, lb_)
    return out


# speedup vs baseline: 1.0938x; 1.0938x over previous
import functools

import jax
import jax.numpy as jnp
from jax import lax
from jax.experimental import pallas as pl
from jax.experimental.pallas import tpu as pltpu
from jax.experimental.pallas import tpu_sc as plsc

N = 4096
NFEAT = 512
TDIM = 128
NCLASS = 64
FANOUT = 4

R = 256


def _gather_sc(table, idx):
    B = idx.shape[0]
    D = table.shape[1]
    info = plsc.get_sparse_core_info()
    nw = info.num_cores * info.num_subcores
    b_per_w = B // nw
    mesh = plsc.VectorSubcoreMesh(core_axis_name="c", subcore_axis_name="s")

    @functools.partial(
        pl.kernel,
        mesh=mesh,
        out_type=jax.ShapeDtypeStruct((B, D), jnp.float32),
        scratch_types=[
            pltpu.VMEM((b_per_w,), jnp.int32),
            pltpu.VMEM((b_per_w, D), jnp.float32),
            pltpu.SemaphoreType.DMA,
        ],
    )
    def k(table_hbm, idx_hbm, out_hbm, idx_v, rows_v, sem):
        wid = lax.axis_index("s") * info.num_cores + lax.axis_index("c")
        base = wid * b_per_w
        pltpu.sync_copy(idx_hbm.at[pl.ds(base, b_per_w)], idx_v)
        pltpu.async_copy(table_hbm.at[idx_v], rows_v, sem).wait()
        pltpu.sync_copy(rows_v, out_hbm.at[pl.ds(base, b_per_w)])

    return k(table, idx)


def _mm(a, b):
    return lax.dot_general(a, b, (((1,), (0,)), ((), ())),
                           preferred_element_type=jnp.float32)


def _mean4_via(scratch, x):
    d = x.shape[1]
    cols = []
    for c in range(d // 128):
        s = scratch[c]
        s[...] = x[:, c * 128:(c + 1) * 128]
        cols.append(s[0::4] + s[1::4] + s[2::4] + s[3::4])
    m = cols[0] if len(cols) == 1 else jnp.concatenate(cols, axis=1)
    return m * 0.25


def _tc_body(x0_r, x1_r, x2_r, xb0_r, e1_r, xb2_r, wt_r,
             w0a_r, w0b_r, b0_r, w1a_r, w1b_r, b1_r,
             wb0a_r, wb0b_r, bb0_r, wb1a_r, wb1b_r, bb1_r,
             lwa_r, lwb_r, lb_r, out_r,
             s1a, s1b, s1c, s1d, s2a, s2b, s2c, s2d,
             sha, shb, se2, sg1):
    relu = jax.nn.relu
    w0a, w0b, b0 = w0a_r[...], w0b_r[...], b0_r[...]
    wb0a, wb0b, bb0 = wb0a_r[...], wb0b_r[...], bb0_r[...]
    wt = wt_r[...]

    x1 = x1_r[...]
    m1 = _mean4_via([s1a, s1b, s1c, s1d], x1)
    h0 = relu(_mm(x0_r[...], w0a) + _mm(m1, w0b) + b0)
    m2 = _mean4_via([s2a, s2b, s2c, s2d], x2_r[...])
    h1 = relu(_mm(x1, w0a) + _mm(m2, w0b) + b0)
    mh1 = _mean4_via([sha, shb], h1)
    o1 = relu(_mm(h0, w1a_r[...]) + _mm(mh1, w1b_r[...]) + b1_r[...])

    e0 = relu(_mm(xb0_r[...], wt))
    me1 = (e1_r[0::4] + e1_r[1::4] + e1_r[2::4] + e1_r[3::4]) * 0.25
    g0 = relu(_mm(e0, wb0a) + _mm(me1, wb0b) + bb0)
    e1 = e1_r[...]
    e2 = relu(_mm(xb2_r[...], wt))
    me2 = _mean4_via([se2], e2)
    g1 = relu(_mm(e1, wb0a) + _mm(me2, wb0b) + bb0)
    mg1 = _mean4_via([sg1], g1)
    o2 = relu(_mm(g0, wb1a_r[...]) + _mm(mg1, wb1b_r[...]) + bb1_r[...])

    z = _mm(o1, lwa_r[...]) + _mm(o2, lwb_r[...]) + lb_r[...]
    zs = z - jnp.max(z, axis=1, keepdims=True)
    out_r[...] = zs - jnp.log(jnp.sum(jnp.exp(zs), axis=1, keepdims=True))


def kernel(x0, x1, x2, x_bi0, x_bi1, x_bi2, table, weight_trans,
           W0, b0, W1, b1, Wb0, bb0, Wb1, bb1, lin_W, lin_b):
    e1 = _gather_sc(table, x_bi1.astype(jnp.int32))

    W0a, W0b = W0[:NFEAT], W0[NFEAT:]
    W1a, W1b = W1[:256], W1[256:]
    Wb0a, Wb0b = Wb0[:TDIM], Wb0[TDIM:]
    Wb1a, Wb1b = Wb1[:128], Wb1[128:]
    lwa, lwb = lin_W[:NCLASS], lin_W[NCLASS:]
    b0_ = b0.reshape(1, -1)
    b1_ = b1.reshape(1, -1)
    bb0_ = bb0.reshape(1, -1)
    bb1_ = bb1.reshape(1, -1)
    lb_ = lin_b.reshape(1, -1)

    nb = N // R
    full = lambda shp: pl.BlockSpec(shp, lambda i: (0, 0))
    row = lambda r, d: pl.BlockSpec((r, d), lambda i: (i, 0))
    out = pl.pallas_call(
        _tc_body,
        grid=(nb,),
        in_specs=[
            row(R, NFEAT),
            row(4 * R, NFEAT),
            row(16 * R, NFEAT),
            row(R, NFEAT),
            row(4 * R, TDIM),
            row(16 * R, NFEAT),
            full((NFEAT, TDIM)),
            full((NFEAT, 256)),
            full((NFEAT, 256)),
            full((1, 256)),
            full((256, NCLASS)),
            full((256, NCLASS)),
            full((1, NCLASS)),
            full((TDIM, 128)),
            full((TDIM, 128)),
            full((1, 128)),
            full((128, 128)),
            full((128, 128)),
            full((1, 128)),
            full((NCLASS, NCLASS)),
            full((128, NCLASS)),
            full((1, NCLASS)),
        ],
        out_specs=row(R, NCLASS),
        out_shape=jax.ShapeDtypeStruct((N, NCLASS), jnp.float32),
        scratch_shapes=[
            pltpu.VMEM((4 * R, 128), jnp.float32),
            pltpu.VMEM((4 * R, 128), jnp.float32),
            pltpu.VMEM((4 * R, 128), jnp.float32),
            pltpu.VMEM((4 * R, 128), jnp.float32),
            pltpu.VMEM((16 * R, 128), jnp.float32),
            pltpu.VMEM((16 * R, 128), jnp.float32),
            pltpu.VMEM((16 * R, 128), jnp.float32),
            pltpu.VMEM((16 * R, 128), jnp.float32),
            pltpu.VMEM((4 * R, 128), jnp.float32),
            pltpu.VMEM((4 * R, 128), jnp.float32),
            pltpu.VMEM((16 * R, 128), jnp.float32),
            pltpu.VMEM((4 * R, 128), jnp.float32),
        ],
    )(x0, x1, x2, x_bi0, e1, x_bi2, weight_trans,
      W0a, W0b, b0_, W1a, W1b, b1_, Wb0a, Wb0b, bb0_,
      Wb1a, Wb1b, bb1_, lwa, lwb, lb_)
    return out
